# Initial kernel scaffold; baseline (speedup 1.0000x reference)
import jax
import jax.numpy as jnp
from jax.experimental import pallas as pl


def kernel(logits, free_agents_num, tasks_num):
    raise NotImplementedError("write your pallas kernel here")


# speedup vs baseline: 2.1888x; 2.1888x over previous
import jax
import jax.numpy as jnp
from jax import lax
from jax.experimental import pallas as pl
from jax.experimental.pallas import tpu as pltpu

_TAU = 1.0
_ITERATIONS = 5
_STRIP = 256


def _sinkhorn_body(a_ref, t_ref, x_hbm, o_hbm, xs, sem_in, sem_out):
    b = pl.program_id(0)
    n = xs.shape[1]
    n_strips = n // _STRIP
    agents = a_ref[b]
    tasks = t_ref[b]

    neg_inf = jnp.float32(-jnp.inf)
    zero = jnp.float32(0.0)
    one = jnp.float32(1.0)

    cols = lax.broadcasted_iota(jnp.int32, (1, n), 1)
    cbias = jnp.where(cols < tasks, zero, neg_inf)
    cmul = jnp.where(cols < tasks, one, zero)
    strip_rows = lax.broadcasted_iota(jnp.int32, (_STRIP, 1), 0)

    def _row_masks(i):
        rows = strip_rows + i * _STRIP
        rbias = jnp.where(rows < agents, zero, neg_inf)
        rmul = jnp.where(rows < agents, one, zero)
        return rbias, rmul

    load = pltpu.make_async_copy(x_hbm.at[b], xs, sem_in)
    load.start()
    load.wait()

    def _row_softmax_pass(i, _):
        rbias, rmul = _row_masks(i)
        blk = xs[pl.ds(i * _STRIP, _STRIP), :]
        xm = (blk + rbias) + cbias
        m = jnp.max(xm, axis=1, keepdims=True)
        m = jnp.where(jnp.isfinite(m), m, zero)
        e = jnp.exp(xm - m) * rmul * cmul
        s = jnp.sum(e, axis=1, keepdims=True)
        denom = jnp.where(s > zero, s, one)
        xs[pl.ds(i * _STRIP, _STRIP), :] = e / denom
        return 0

    def _col_max_pass(i, mcol):
        rbias, _ = _row_masks(i)
        blk = xs[pl.ds(i * _STRIP, _STRIP), :]
        xm = (blk + rbias) + cbias
        return jnp.maximum(mcol, jnp.max(xm, axis=0, keepdims=True))

    def _col_exp_pass(i, scol):
        rbias, rmul = _row_masks(i)
        blk = xs[pl.ds(i * _STRIP, _STRIP), :]
        xm = (blk + rbias) + cbias
        e = jnp.exp(xm - mcol_fixed[0]) * rmul * cmul
        xs[pl.ds(i * _STRIP, _STRIP), :] = e
        return scol + jnp.sum(e, axis=0, keepdims=True)

    def _col_div_pass(i, _):
        blk = xs[pl.ds(i * _STRIP, _STRIP), :]
        xs[pl.ds(i * _STRIP, _STRIP), :] = blk / denom_fixed[0]
        return 0

    for _ in range(_ITERATIONS):
        lax.fori_loop(0, n_strips, _row_softmax_pass, 0)
        mcol = lax.fori_loop(
            0, n_strips, _col_max_pass,
            jnp.full((1, n), neg_inf, dtype=jnp.float32))
        mcol = jnp.where(jnp.isfinite(mcol), mcol, zero)
        mcol_fixed = [mcol]
        scol = lax.fori_loop(
            0, n_strips, _col_exp_pass,
            jnp.zeros((1, n), dtype=jnp.float32))
        denom_fixed = [jnp.where(scol > zero, scol, one)]
        lax.fori_loop(0, n_strips, _col_div_pass, 0)

    store = pltpu.make_async_copy(xs, o_hbm.at[b], sem_out)
    store.start()
    store.wait()


def kernel(logits, free_agents_num, tasks_num):
    b, n, _ = logits.shape
    grid_spec = pltpu.PrefetchScalarGridSpec(
        num_scalar_prefetch=2,
        grid=(b,),
        in_specs=[pl.BlockSpec(memory_space=pl.ANY)],
        out_specs=pl.BlockSpec(memory_space=pl.ANY),
        scratch_shapes=[
            pltpu.VMEM((n, n), jnp.float32),
            pltpu.SemaphoreType.DMA,
            pltpu.SemaphoreType.DMA,
        ],
    )
    return pl.pallas_call(
        _sinkhorn_body,
        grid_spec=grid_spec,
        out_shape=jax.ShapeDtypeStruct((b, n, n), jnp.float32),
    )(free_agents_num, tasks_num, logits)


# speedup vs baseline: 4.8590x; 2.2200x over previous
import jax
import jax.numpy as jnp
from jax import lax
from jax.experimental import pallas as pl
from jax.experimental.pallas import tpu as pltpu

_TAU = 1.0
_ITERATIONS = 5
_STRIP = 256


def _sinkhorn_body(a_ref, t_ref, x_hbm, o_hbm, xs, sem_in, sem_out):
    b = pl.program_id(0)
    n = xs.shape[1]
    n_strips = n // _STRIP
    agents = a_ref[b]
    tasks = t_ref[b]
    n_valid = lax.div(agents + (_STRIP - 1), _STRIP)

    neg_inf = jnp.float32(-jnp.inf)
    zero = jnp.float32(0.0)
    one = jnp.float32(1.0)

    cols = lax.broadcasted_iota(jnp.int32, (1, n), 1)
    cbias = jnp.where(cols < tasks, zero, neg_inf)
    cmul = jnp.where(cols < tasks, one, zero)
    strip_rows = lax.broadcasted_iota(jnp.int32, (_STRIP, 1), 0)

    load = pltpu.make_async_copy(x_hbm.at[b], xs, sem_in)
    load.start()
    load.wait()

    def _zero_pass(i, _):
        xs[pl.ds(i * _STRIP, _STRIP), :] = jnp.zeros((_STRIP, n), jnp.float32)
        return 0

    def _rowsm_first(i, _):
        rows = strip_rows + i * _STRIP
        rbias = jnp.where(rows < agents, zero, neg_inf)
        blk = xs[pl.ds(i * _STRIP, _STRIP), :]
        xm = (blk + rbias) + cbias
        m = jnp.max(xm, axis=1, keepdims=True)
        m = jnp.where(jnp.isfinite(m), m, zero)
        e = jnp.exp(xm - m)
        s = jnp.sum(e, axis=1, keepdims=True)
        denom = jnp.where(s > zero, s, one)
        xs[pl.ds(i * _STRIP, _STRIP), :] = e / denom
        return 0

    def _colexp_pass(i, scol):
        rows = strip_rows + i * _STRIP
        rmul = jnp.where(rows < agents, one, zero)
        blk = xs[pl.ds(i * _STRIP, _STRIP), :]
        e = jnp.exp(blk) * rmul * cmul
        xs[pl.ds(i * _STRIP, _STRIP), :] = e
        return scol + jnp.sum(e, axis=0, keepdims=True)

    def _make_rowsm(dcol):
        def _rowsm(i, _):
            rows = strip_rows + i * _STRIP
            rmul = jnp.where(rows < agents, one, zero)
            blk = xs[pl.ds(i * _STRIP, _STRIP), :] / dcol
            e = jnp.exp(blk) * rmul * cmul
            s = jnp.sum(e, axis=1, keepdims=True)
            denom = jnp.where(s > zero, s, one)
            xs[pl.ds(i * _STRIP, _STRIP), :] = e / denom
            return 0
        return _rowsm

    def _make_final_div(dcol):
        def _div(i, _):
            xs[pl.ds(i * _STRIP, _STRIP), :] = (
                xs[pl.ds(i * _STRIP, _STRIP), :] / dcol)
            return 0
        return _div

    lax.fori_loop(n_valid, n_strips, _zero_pass, 0)
    lax.fori_loop(0, n_valid, _rowsm_first, 0)
    scol = lax.fori_loop(0, n_valid, _colexp_pass,
                         jnp.zeros((1, n), dtype=jnp.float32))
    dcol = jnp.where(scol > zero, scol, one)
    for _ in range(_ITERATIONS - 1):
        lax.fori_loop(0, n_valid, _make_rowsm(dcol), 0)
        scol = lax.fori_loop(0, n_valid, _colexp_pass,
                             jnp.zeros((1, n), dtype=jnp.float32))
        dcol = jnp.where(scol > zero, scol, one)
    lax.fori_loop(0, n_valid, _make_final_div(dcol), 0)

    store = pltpu.make_async_copy(xs, o_hbm.at[b], sem_out)
    store.start()
    store.wait()


def kernel(logits, free_agents_num, tasks_num):
    b, n, _ = logits.shape
    grid_spec = pltpu.PrefetchScalarGridSpec(
        num_scalar_prefetch=2,
        grid=(b,),
        in_specs=[pl.BlockSpec(memory_space=pl.ANY)],
        out_specs=pl.BlockSpec(memory_space=pl.ANY),
        scratch_shapes=[
            pltpu.VMEM((n, n), jnp.float32),
            pltpu.SemaphoreType.DMA,
            pltpu.SemaphoreType.DMA,
        ],
    )
    return pl.pallas_call(
        _sinkhorn_body,
        grid_spec=grid_spec,
        out_shape=jax.ShapeDtypeStruct((b, n, n), jnp.float32),
    )(free_agents_num, tasks_num, logits)


# speedup vs baseline: 5.7637x; 1.1862x over previous
import jax
import jax.numpy as jnp
from jax import lax
from jax.experimental import pallas as pl
from jax.experimental.pallas import tpu as pltpu

_TAU = 1.0
_ITERATIONS = 5
_STRIP = 256
_CHUNK = 512


def _sinkhorn_body(a_ref, t_ref, x_hbm, o_hbm, xs, drow_ref, scol_ref,
                   sem_in, sem_out):
    b = pl.program_id(0)
    n = xs.shape[1]
    S, C = _STRIP, _CHUNK
    n_strips = n // S
    n_chunks = n // C
    agents = a_ref[b]
    tasks = t_ref[b]
    nv = lax.div(agents + (S - 1), S)
    cv = lax.div(tasks + (C - 1), C)

    neg_inf = jnp.float32(-jnp.inf)
    zero = jnp.float32(0.0)
    one = jnp.float32(1.0)

    strip_rows = lax.broadcasted_iota(jnp.int32, (S, 1), 0)
    chunk_cols = lax.broadcasted_iota(jnp.int32, (1, C), 1)

    def _cbias(c):
        return jnp.where(chunk_cols + c * C < tasks, zero, neg_inf)

    def _cmul(c):
        return jnp.where(chunk_cols + c * C < tasks, one, zero)

    def _strip_load(i):
        return pltpu.make_async_copy(
            x_hbm.at[b, pl.ds(i * S, S), :], xs.at[pl.ds(i * S, S), :], sem_in)

    def _strip_store(i):
        return pltpu.make_async_copy(
            xs.at[pl.ds(i * S, S), :], o_hbm.at[b, pl.ds(i * S, S), :], sem_out)

    lax.fori_loop(0, nv, lambda i, _: (_strip_load(i).start(), 0)[1], 0)

    def _zero_strip(i, _):
        xs[pl.ds(i * S, S), :] = jnp.zeros((S, n), jnp.float32)
        _strip_store(i).start()
        return 0
    lax.fori_loop(nv, n_strips, _zero_strip, 0)

    lax.fori_loop(0, nv, lambda i, _: (_strip_load(i).wait(), 0)[1], 0)

    def _zero_chunks(i, _):
        def _zc(c, _):
            xs[pl.ds(i * S, S), pl.ds(c * C, C)] = jnp.zeros((S, C), jnp.float32)
            return 0
        return lax.fori_loop(cv, n_chunks, _zc, 0)
    lax.fori_loop(0, nv, _zero_chunks, 0)

    def _pass1(i, _):
        rows = strip_rows + i * S
        rbias = jnp.where(rows < agents, zero, neg_inf)

        def _mx(c, m):
            blk = xs[pl.ds(i * S, S), pl.ds(c * C, C)]
            xm = (blk + rbias) + _cbias(c)
            return jnp.maximum(m, jnp.max(xm, axis=1, keepdims=True))
        m = lax.fori_loop(0, cv, _mx, jnp.full((S, 1), neg_inf, jnp.float32))
        m = jnp.where(jnp.isfinite(m), m, zero)

        def _ex(c, s):
            blk = xs[pl.ds(i * S, S), pl.ds(c * C, C)]
            e = jnp.exp(((blk + rbias) + _cbias(c)) - m)
            xs[pl.ds(i * S, S), pl.ds(c * C, C)] = e
            return s + jnp.sum(e, axis=1, keepdims=True)
        s = lax.fori_loop(0, cv, _ex, jnp.zeros((S, 1), jnp.float32))
        drow_ref[pl.ds(i * S, S), :] = jnp.where(s > zero, s, one)
        return 0
    lax.fori_loop(0, nv, _pass1, 0)

    def _colexp_pass():
        scol_ref[:, :] = jnp.zeros((1, n), jnp.float32)

        def _strip(i, _):
            rows = strip_rows + i * S
            rmul = jnp.where(rows < agents, one, zero)
            dr = drow_ref[pl.ds(i * S, S), :]

            def _c(c, _):
                blk = xs[pl.ds(i * S, S), pl.ds(c * C, C)] / dr
                e = jnp.exp(blk) * rmul * _cmul(c)
                xs[pl.ds(i * S, S), pl.ds(c * C, C)] = e
                scol_ref[:, pl.ds(c * C, C)] = (
                    scol_ref[:, pl.ds(c * C, C)]
                    + jnp.sum(e, axis=0, keepdims=True))
                return 0
            return lax.fori_loop(0, cv, _c, 0)
        lax.fori_loop(0, nv, _strip, 0)

    def _dcol(c):
        s = scol_ref[:, pl.ds(c * C, C)]
        return jnp.where(s > zero, s, one)

    def _rowexp(i, _):
        rows = strip_rows + i * S
        rmul = jnp.where(rows < agents, one, zero)

        def _c(c, s):
            blk = xs[pl.ds(i * S, S), pl.ds(c * C, C)] / _dcol(c)
            e = jnp.exp(blk) * rmul * _cmul(c)
            xs[pl.ds(i * S, S), pl.ds(c * C, C)] = e
            return s + jnp.sum(e, axis=1, keepdims=True)
        s = lax.fori_loop(0, cv, _c, jnp.zeros((S, 1), jnp.float32))
        drow_ref[pl.ds(i * S, S), :] = jnp.where(s > zero, s, one)
        return 0

    def _final(i, _):
        def _c(c, _):
            blk = xs[pl.ds(i * S, S), pl.ds(c * C, C)]
            xs[pl.ds(i * S, S), pl.ds(c * C, C)] = blk / _dcol(c)
            return 0
        lax.fori_loop(0, cv, _c, 0)
        _strip_store(i).start()
        return 0

    _colexp_pass()
    for _ in range(_ITERATIONS - 1):
        lax.fori_loop(0, nv, _rowexp, 0)
        _colexp_pass()
    lax.fori_loop(0, nv, _final, 0)

    lax.fori_loop(0, n_strips, lambda i, _: (_strip_store(i).wait(), 0)[1], 0)


def kernel(logits, free_agents_num, tasks_num):
    b, n, _ = logits.shape
    grid_spec = pltpu.PrefetchScalarGridSpec(
        num_scalar_prefetch=2,
        grid=(b,),
        in_specs=[pl.BlockSpec(memory_space=pl.ANY)],
        out_specs=pl.BlockSpec(memory_space=pl.ANY),
        scratch_shapes=[
            pltpu.VMEM((n, n), jnp.float32),
            pltpu.VMEM((n, 1), jnp.float32),
            pltpu.VMEM((1, n), jnp.float32),
            pltpu.SemaphoreType.DMA,
            pltpu.SemaphoreType.DMA,
        ],
    )
    return pl.pallas_call(
        _sinkhorn_body,
        grid_spec=grid_spec,
        out_shape=jax.ShapeDtypeStruct((b, n, n), jnp.float32),
    )(free_agents_num, tasks_num, logits)


# speedup vs baseline: 5.8498x; 1.0149x over previous
import jax
import jax.numpy as jnp
from jax import lax
from jax.experimental import pallas as pl
from jax.experimental.pallas import tpu as pltpu

_TAU = 1.0
_ITERATIONS = 5
_STRIP = 256
_CHUNK = 512


def _sinkhorn_body(a_ref, t_ref, x_hbm, o_hbm, xs, drow_ref, scol_ref,
                   sem_in, sem_out):
    b = pl.program_id(0)
    n = xs.shape[1]
    S, C = _STRIP, _CHUNK
    n_strips = n // S
    n_chunks = n // C
    agents = a_ref[b]
    tasks = t_ref[b]
    nv = lax.div(agents + (S - 1), S)
    cv = lax.div(tasks + (C - 1), C)

    neg_inf = jnp.float32(-jnp.inf)
    zero = jnp.float32(0.0)
    one = jnp.float32(1.0)

    strip_rows = lax.broadcasted_iota(jnp.int32, (S, 1), 0)
    chunk_cols = lax.broadcasted_iota(jnp.int32, (1, C), 1)

    def _cbias(c):
        return jnp.where(chunk_cols + c * C < tasks, zero, neg_inf)

    def _cmul(c):
        return jnp.where(chunk_cols + c * C < tasks, one, zero)

    def _strip_load(i):
        return pltpu.make_async_copy(
            x_hbm.at[b, pl.ds(i * S, S), :], xs.at[pl.ds(i * S, S), :], sem_in)

    def _strip_store(i):
        return pltpu.make_async_copy(
            xs.at[pl.ds(i * S, S), :], o_hbm.at[b, pl.ds(i * S, S), :], sem_out)

    lax.fori_loop(0, nv, lambda i, _: (_strip_load(i).start(), 0)[1], 0)

    def _zero_strip(i, _):
        xs[pl.ds(i * S, S), :] = jnp.zeros((S, n), jnp.float32)
        _strip_store(i).start()
        return 0
    lax.fori_loop(nv, n_strips, _zero_strip, 0)

    lax.fori_loop(0, nv, lambda i, _: (_strip_load(i).wait(), 0)[1], 0)

    def _zero_chunks(i, _):
        def _zc(c, _):
            xs[pl.ds(i * S, S), pl.ds(c * C, C)] = jnp.zeros((S, C), jnp.float32)
            return 0
        return lax.fori_loop(cv, n_chunks, _zc, 0)
    lax.fori_loop(0, nv, _zero_chunks, 0)

    def _pass1(i, _):
        rows = strip_rows + i * S
        rbias = jnp.where(rows < agents, zero, neg_inf)

        def _mx(c, m):
            blk = xs[pl.ds(i * S, S), pl.ds(c * C, C)]
            xm = (blk + rbias) + _cbias(c)
            return jnp.maximum(m, jnp.max(xm, axis=1, keepdims=True))
        m = lax.fori_loop(0, cv, _mx, jnp.full((S, 1), neg_inf, jnp.float32))
        m = jnp.where(jnp.isfinite(m), m, zero)

        def _ex(c, s):
            blk = xs[pl.ds(i * S, S), pl.ds(c * C, C)]
            e = jnp.exp(((blk + rbias) + _cbias(c)) - m)
            xs[pl.ds(i * S, S), pl.ds(c * C, C)] = e
            return s + jnp.sum(e, axis=1, keepdims=True)
        s = lax.fori_loop(0, cv, _ex, jnp.zeros((S, 1), jnp.float32))
        drow_ref[pl.ds(i * S, S), :] = jnp.where(s > zero, one / s, one)
        return 0
    lax.fori_loop(0, nv, _pass1, 0)

    def _colexp_pass():
        scol_ref[:, :] = jnp.zeros((1, n), jnp.float32)

        def _strip(i, _):
            rows = strip_rows + i * S
            rmul = jnp.where(rows < agents, one, zero)
            dr = drow_ref[pl.ds(i * S, S), :]

            def _c(c, _):
                blk = xs[pl.ds(i * S, S), pl.ds(c * C, C)] * dr
                e = jnp.exp(blk) * rmul * _cmul(c)
                xs[pl.ds(i * S, S), pl.ds(c * C, C)] = e
                scol_ref[:, pl.ds(c * C, C)] = (
                    scol_ref[:, pl.ds(c * C, C)]
                    + jnp.sum(e, axis=0, keepdims=True))
                return 0
            return lax.fori_loop(0, cv, _c, 0)
        lax.fori_loop(0, nv, _strip, 0)

    def _rdcol(c):
        s = scol_ref[:, pl.ds(c * C, C)]
        return jnp.where(s > zero, one / s, one)

    def _rowexp(i, _):
        rows = strip_rows + i * S
        rmul = jnp.where(rows < agents, one, zero)

        def _c(c, s):
            blk = xs[pl.ds(i * S, S), pl.ds(c * C, C)] * _rdcol(c)
            e = jnp.exp(blk) * rmul * _cmul(c)
            xs[pl.ds(i * S, S), pl.ds(c * C, C)] = e
            return s + jnp.sum(e, axis=1, keepdims=True)
        s = lax.fori_loop(0, cv, _c, jnp.zeros((S, 1), jnp.float32))
        drow_ref[pl.ds(i * S, S), :] = jnp.where(s > zero, one / s, one)
        return 0

    def _final(i, _):
        def _c(c, _):
            blk = xs[pl.ds(i * S, S), pl.ds(c * C, C)]
            xs[pl.ds(i * S, S), pl.ds(c * C, C)] = blk * _rdcol(c)
            return 0
        lax.fori_loop(0, cv, _c, 0)
        _strip_store(i).start()
        return 0

    _colexp_pass()
    for _ in range(_ITERATIONS - 1):
        lax.fori_loop(0, nv, _rowexp, 0)
        _colexp_pass()
    lax.fori_loop(0, nv, _final, 0)

    lax.fori_loop(0, n_strips, lambda i, _: (_strip_store(i).wait(), 0)[1], 0)


def kernel(logits, free_agents_num, tasks_num):
    b, n, _ = logits.shape
    grid_spec = pltpu.PrefetchScalarGridSpec(
        num_scalar_prefetch=2,
        grid=(b,),
        in_specs=[pl.BlockSpec(memory_space=pl.ANY)],
        out_specs=pl.BlockSpec(memory_space=pl.ANY),
        scratch_shapes=[
            pltpu.VMEM((n, n), jnp.float32),
            pltpu.VMEM((n, 1), jnp.float32),
            pltpu.VMEM((1, n), jnp.float32),
            pltpu.SemaphoreType.DMA,
            pltpu.SemaphoreType.DMA,
        ],
    )
    return pl.pallas_call(
        _sinkhorn_body,
        grid_spec=grid_spec,
        out_shape=jax.ShapeDtypeStruct((b, n, n), jnp.float32),
    )(free_agents_num, tasks_num, logits)


# speedup vs baseline: 7.7788x; 1.3298x over previous
import jax
import jax.numpy as jnp
from jax import lax
from jax.experimental import pallas as pl
from jax.experimental.pallas import tpu as pltpu

_TAU = 1.0
_ITERATIONS = 5
_STRIP = 256
_CHUNK = 512
_NBUF = 3


def _sinkhorn_body(a_ref, t_ref, x_hbm, o_hbm, xs3, drow_ref, scol_ref,
                   sem_in, sem_out):
    b = pl.program_id(0)
    nb = pl.num_programs(0)
    n = xs3.shape[2]
    S, C = _STRIP, _CHUNK
    n_strips = n // S
    n_chunks = n // C
    p = lax.rem(b, _NBUF)

    agents = a_ref[b]
    tasks = t_ref[b]
    nv = lax.div(agents + (S - 1), S)
    cv = lax.div(tasks + (C - 1), C)

    neg_inf = jnp.float32(-jnp.inf)
    zero = jnp.float32(0.0)
    one = jnp.float32(1.0)

    strip_rows = lax.broadcasted_iota(jnp.int32, (S, 1), 0)
    chunk_cols = lax.broadcasted_iota(jnp.int32, (1, C), 1)

    def _cbias(c):
        return jnp.where(chunk_cols + c * C < tasks, zero, neg_inf)

    def _cmul(c):
        return jnp.where(chunk_cols + c * C < tasks, one, zero)

    def _load_cp(sample, buf, i):
        return pltpu.make_async_copy(
            x_hbm.at[sample, pl.ds(i * S, S), :],
            xs3.at[buf, pl.ds(i * S, S), :], sem_in.at[buf])

    def _store_cp(buf, i, sample):
        return pltpu.make_async_copy(
            xs3.at[buf, pl.ds(i * S, S), :],
            o_hbm.at[sample, pl.ds(i * S, S), :], sem_out.at[buf])

    def _issue_loads(sample, buf):
        nvs = lax.div(a_ref[sample] + (S - 1), S)
        lax.fori_loop(0, nvs, lambda i, _: (_load_cp(sample, buf, i).start(), 0)[1], 0)

    def _drain_stores(buf):
        lax.fori_loop(0, n_strips,
                      lambda i, _: (_store_cp(buf, 0, 0).wait(), 0)[1], 0)

    @pl.when(b == 0)
    def _():
        _issue_loads(0, 0)

    lax.fori_loop(0, nv, lambda i, _: (_load_cp(b, p, i).wait(), 0)[1], 0)

    @pl.when(b < nb - 1)
    def _():
        nxt_buf = lax.rem(b + 1, _NBUF)

        @pl.when(b >= _NBUF - 1)
        def _():
            _drain_stores(nxt_buf)
        _issue_loads(b + 1, nxt_buf)

    def _zero_strip(i, _):
        xs3[p, pl.ds(i * S, S), :] = jnp.zeros((S, n), jnp.float32)
        _store_cp(p, i, b).start()
        return 0
    lax.fori_loop(nv, n_strips, _zero_strip, 0)

    def _zero_chunks(i, _):
        def _zc(c, _):
            xs3[p, pl.ds(i * S, S), pl.ds(c * C, C)] = jnp.zeros(
                (S, C), jnp.float32)
            return 0
        return lax.fori_loop(cv, n_chunks, _zc, 0)
    lax.fori_loop(0, nv, _zero_chunks, 0)

    def _pass1(i, _):
        rows = strip_rows + i * S
        rbias = jnp.where(rows < agents, zero, neg_inf)

        def _mx(c, m):
            blk = xs3[p, pl.ds(i * S, S), pl.ds(c * C, C)]
            xm = (blk + rbias) + _cbias(c)
            return jnp.maximum(m, jnp.max(xm, axis=1, keepdims=True))
        m = lax.fori_loop(0, cv, _mx, jnp.full((S, 1), neg_inf, jnp.float32))
        m = jnp.where(jnp.isfinite(m), m, zero)

        def _ex(c, s):
            blk = xs3[p, pl.ds(i * S, S), pl.ds(c * C, C)]
            e = jnp.exp(((blk + rbias) + _cbias(c)) - m)
            xs3[p, pl.ds(i * S, S), pl.ds(c * C, C)] = e
            return s + jnp.sum(e, axis=1, keepdims=True)
        s = lax.fori_loop(0, cv, _ex, jnp.zeros((S, 1), jnp.float32))
        drow_ref[pl.ds(i * S, S), :] = jnp.where(s > zero, one / s, one)
        return 0
    lax.fori_loop(0, nv, _pass1, 0)

    def _colexp_pass():
        scol_ref[:, :] = jnp.zeros((1, n), jnp.float32)

        def _strip(i, _):
            rows = strip_rows + i * S
            rmul = jnp.where(rows < agents, one, zero)
            dr = drow_ref[pl.ds(i * S, S), :]

            def _c(c, _):
                blk = xs3[p, pl.ds(i * S, S), pl.ds(c * C, C)] * dr
                e = jnp.exp(blk) * rmul * _cmul(c)
                xs3[p, pl.ds(i * S, S), pl.ds(c * C, C)] = e
                scol_ref[:, pl.ds(c * C, C)] = (
                    scol_ref[:, pl.ds(c * C, C)]
                    + jnp.sum(e, axis=0, keepdims=True))
                return 0
            return lax.fori_loop(0, cv, _c, 0)
        lax.fori_loop(0, nv, _strip, 0)

    def _rdcol(c):
        s = scol_ref[:, pl.ds(c * C, C)]
        return jnp.where(s > zero, one / s, one)

    def _rowexp(i, _):
        rows = strip_rows + i * S
        rmul = jnp.where(rows < agents, one, zero)

        def _c(c, s):
            blk = xs3[p, pl.ds(i * S, S), pl.ds(c * C, C)] * _rdcol(c)
            e = jnp.exp(blk) * rmul * _cmul(c)
            xs3[p, pl.ds(i * S, S), pl.ds(c * C, C)] = e
            return s + jnp.sum(e, axis=1, keepdims=True)
        s = lax.fori_loop(0, cv, _c, jnp.zeros((S, 1), jnp.float32))
        drow_ref[pl.ds(i * S, S), :] = jnp.where(s > zero, one / s, one)
        return 0

    def _final(i, _):
        def _c(c, _):
            blk = xs3[p, pl.ds(i * S, S), pl.ds(c * C, C)]
            xs3[p, pl.ds(i * S, S), pl.ds(c * C, C)] = blk * _rdcol(c)
            return 0
        lax.fori_loop(0, cv, _c, 0)
        _store_cp(p, i, b).start()
        return 0

    _colexp_pass()
    for _ in range(_ITERATIONS - 1):
        lax.fori_loop(0, nv, _rowexp, 0)
        _colexp_pass()
    lax.fori_loop(0, nv, _final, 0)

    @pl.when(b == nb - 1)
    def _():
        for k in range(_NBUF):
            _drain_stores(k)


def kernel(logits, free_agents_num, tasks_num):
    b, n, _ = logits.shape
    grid_spec = pltpu.PrefetchScalarGridSpec(
        num_scalar_prefetch=2,
        grid=(b,),
        in_specs=[pl.BlockSpec(memory_space=pl.ANY)],
        out_specs=pl.BlockSpec(memory_space=pl.ANY),
        scratch_shapes=[
            pltpu.VMEM((_NBUF, n, n), jnp.float32),
            pltpu.VMEM((n, 1), jnp.float32),
            pltpu.VMEM((1, n), jnp.float32),
            pltpu.SemaphoreType.DMA((_NBUF,)),
            pltpu.SemaphoreType.DMA((_NBUF,)),
        ],
    )
    return pl.pallas_call(
        _sinkhorn_body,
        grid_spec=grid_spec,
        out_shape=jax.ShapeDtypeStruct((b, n, n), jnp.float32),
    )(free_agents_num, tasks_num, logits)


# speedup vs baseline: 9.1838x; 1.1806x over previous
import jax
import jax.numpy as jnp
from jax import lax
from jax.experimental import pallas as pl
from jax.experimental.pallas import tpu as pltpu

_TAU = 1.0
_ITERATIONS = 5
_STRIP = 256
_CHUNK = 512
_NBUF = 3
_LOG2E = 1.4426950408889634


def _sinkhorn_body(a_ref, t_ref, x_hbm, o_hbm, xs3, drow_ref, scol_ref,
                   sem_in, sem_out):
    b = pl.program_id(0)
    nb = pl.num_programs(0)
    n = xs3.shape[2]
    S, C = _STRIP, _CHUNK
    n_strips = n // S
    n_chunks = n // C
    p = lax.rem(b, _NBUF)

    agents = a_ref[b]
    tasks = t_ref[b]
    nv = lax.div(agents + (S - 1), S)
    cv = lax.div(tasks + (C - 1), C)
    last_i = jnp.maximum(nv - 1, 0)
    last_c = jnp.maximum(cv - 1, 0)

    neg_inf = jnp.float32(-jnp.inf)
    zero = jnp.float32(0.0)
    one = jnp.float32(1.0)
    lg2e = jnp.float32(_LOG2E)

    strip_rows = lax.broadcasted_iota(jnp.int32, (S, 1), 0)
    chunk_cols = lax.broadcasted_iota(jnp.int32, (1, C), 1)

    def _rbias(i):
        return jnp.where(strip_rows + i * S < agents, zero, neg_inf)

    def _cbias(c):
        return jnp.where(chunk_cols + c * C < tasks, zero, neg_inf)

    def _load_cp(sample, buf, i):
        return pltpu.make_async_copy(
            x_hbm.at[sample, pl.ds(i * S, S), :],
            xs3.at[buf, pl.ds(i * S, S), :], sem_in.at[buf])

    def _store_cp(buf, i, sample):
        return pltpu.make_async_copy(
            xs3.at[buf, pl.ds(i * S, S), :],
            o_hbm.at[sample, pl.ds(i * S, S), :], sem_out.at[buf])

    def _issue_loads(sample, buf):
        nvs = lax.div(a_ref[sample] + (S - 1), S)
        lax.fori_loop(0, nvs, lambda i, _: (_load_cp(sample, buf, i).start(), 0)[1], 0)

    def _drain_stores(buf):
        lax.fori_loop(0, n_strips,
                      lambda i, _: (_store_cp(buf, 0, 0).wait(), 0)[1], 0)

    @pl.when(b == 0)
    def _():
        _issue_loads(0, 0)

    lax.fori_loop(0, nv, lambda i, _: (_load_cp(b, p, i).wait(), 0)[1], 0)

    @pl.when(b < nb - 1)
    def _():
        nxt_buf = lax.rem(b + 1, _NBUF)

        @pl.when(b >= _NBUF - 1)
        def _():
            _drain_stores(nxt_buf)
        _issue_loads(b + 1, nxt_buf)

    def _zero_strip(i, _):
        xs3[p, pl.ds(i * S, S), :] = jnp.zeros((S, n), jnp.float32)
        _store_cp(p, i, b).start()
        return 0
    lax.fori_loop(nv, n_strips, _zero_strip, 0)

    def _zero_chunks(i, _):
        def _zc(c, _):
            xs3[p, pl.ds(i * S, S), pl.ds(c * C, C)] = jnp.zeros(
                (S, C), jnp.float32)
            return 0
        return lax.fori_loop(cv, n_chunks, _zc, 0)
    lax.fori_loop(0, nv, _zero_chunks, 0)

    def _pass1_strip(i, rbias):
        def _xm(c, cbias):
            blk = xs3[p, pl.ds(i * S, S), pl.ds(c * C, C)] * lg2e
            if rbias is not None:
                blk = blk + rbias
            if cbias is not None:
                blk = blk + cbias
            return blk

        def _mx(c, m):
            return jnp.maximum(m, jnp.max(_xm(c, None), axis=1, keepdims=True))
        m = lax.fori_loop(0, cv - 1, _mx,
                          jnp.full((S, 1), neg_inf, jnp.float32))
        m = jnp.maximum(m, jnp.max(_xm(last_c, _cbias(last_c)),
                                   axis=1, keepdims=True))
        m = jnp.where(jnp.isfinite(m), m, zero)

        def _ex(c, s):
            e = jnp.exp2(_xm(c, None) - m)
            xs3[p, pl.ds(i * S, S), pl.ds(c * C, C)] = e
            return s + jnp.sum(e, axis=1, keepdims=True)
        s = lax.fori_loop(0, cv - 1, _ex, jnp.zeros((S, 1), jnp.float32))
        e = jnp.exp2(_xm(last_c, _cbias(last_c)) - m)
        xs3[p, pl.ds(i * S, S), pl.ds(last_c * C, C)] = e
        s = s + jnp.sum(e, axis=1, keepdims=True)
        drow_ref[pl.ds(i * S, S), :] = jnp.where(s > zero, lg2e / s, lg2e)

    lax.fori_loop(0, nv - 1, lambda i, _: (_pass1_strip(i, None), 0)[1], 0)
    _pass1_strip(last_i, _rbias(last_i))

    def _colexp_strip(i, rbias):
        dr = drow_ref[pl.ds(i * S, S), :]

        def _body(c, cbias):
            blk = xs3[p, pl.ds(i * S, S), pl.ds(c * C, C)] * dr
            if rbias is not None:
                blk = blk + rbias
            if cbias is not None:
                blk = blk + cbias
            e = jnp.exp2(blk)
            xs3[p, pl.ds(i * S, S), pl.ds(c * C, C)] = e
            scol_ref[:, pl.ds(c * C, C)] = (
                scol_ref[:, pl.ds(c * C, C)]
                + jnp.sum(e, axis=0, keepdims=True))

        lax.fori_loop(0, cv - 1, lambda c, _: (_body(c, None), 0)[1], 0)
        _body(last_c, _cbias(last_c))

    def _colexp_pass():
        scol_ref[:, :] = jnp.zeros((1, n), jnp.float32)
        lax.fori_loop(0, nv - 1, lambda i, _: (_colexp_strip(i, None), 0)[1], 0)
        _colexp_strip(last_i, _rbias(last_i))

    def _rdcol_scaled(c):
        s = scol_ref[:, pl.ds(c * C, C)]
        return jnp.where(s > zero, lg2e / s, lg2e)

    def _rdcol_true(c):
        s = scol_ref[:, pl.ds(c * C, C)]
        return jnp.where(s > zero, one / s, one)

    def _rowexp_strip(i, rbias):
        def _body(c, cbias, s):
            blk = xs3[p, pl.ds(i * S, S), pl.ds(c * C, C)] * _rdcol_scaled(c)
            if rbias is not None:
                blk = blk + rbias
            if cbias is not None:
                blk = blk + cbias
            e = jnp.exp2(blk)
            xs3[p, pl.ds(i * S, S), pl.ds(c * C, C)] = e
            return s + jnp.sum(e, axis=1, keepdims=True)

        s = lax.fori_loop(0, cv - 1, lambda c, s: _body(c, None, s),
                          jnp.zeros((S, 1), jnp.float32))
        s = _body(last_c, _cbias(last_c), s)
        drow_ref[pl.ds(i * S, S), :] = jnp.where(s > zero, lg2e / s, lg2e)

    def _rowexp_pass():
        lax.fori_loop(0, nv - 1, lambda i, _: (_rowexp_strip(i, None), 0)[1], 0)
        _rowexp_strip(last_i, _rbias(last_i))

    def _final(i, _):
        def _c(c, _):
            blk = xs3[p, pl.ds(i * S, S), pl.ds(c * C, C)]
            xs3[p, pl.ds(i * S, S), pl.ds(c * C, C)] = blk * _rdcol_true(c)
            return 0
        lax.fori_loop(0, cv, _c, 0)
        _store_cp(p, i, b).start()
        return 0

    _colexp_pass()
    for _ in range(_ITERATIONS - 1):
        _rowexp_pass()
        _colexp_pass()
    lax.fori_loop(0, nv, _final, 0)

    @pl.when(b == nb - 1)
    def _():
        for k in range(_NBUF):
            _drain_stores(k)


def kernel(logits, free_agents_num, tasks_num):
    b, n, _ = logits.shape
    grid_spec = pltpu.PrefetchScalarGridSpec(
        num_scalar_prefetch=2,
        grid=(b,),
        in_specs=[pl.BlockSpec(memory_space=pl.ANY)],
        out_specs=pl.BlockSpec(memory_space=pl.ANY),
        scratch_shapes=[
            pltpu.VMEM((_NBUF, n, n), jnp.float32),
            pltpu.VMEM((n, 1), jnp.float32),
            pltpu.VMEM((1, n), jnp.float32),
            pltpu.SemaphoreType.DMA((_NBUF,)),
            pltpu.SemaphoreType.DMA((_NBUF,)),
        ],
    )
    return pl.pallas_call(
        _sinkhorn_body,
        grid_spec=grid_spec,
        out_shape=jax.ShapeDtypeStruct((b, n, n), jnp.float32),
    )(free_agents_num, tasks_num, logits)
